# baseline (speedup 1.0000x reference)
import jax
import jax.numpy as jnp
from jax import lax
from jax.experimental import pallas as pl
from jax.experimental.pallas import tpu as pltpu
from jax.experimental.pallas import tpu_sc as plsc

D = 64
B = 16384

NW = 32
BPW = B // NW
HALF = BPW // 2
NG = HALF // 16


def _row_copies(ut3_hbm, at3_hbm, urows_v, arows_v, uidx_v, aidx_v, hb, g, sem):
    u16 = uidx_v[pl.ds(hb + g * 16, 16)]
    a16 = aidx_v[pl.ds(hb + g * 16, 16)]
    copies = []
    for l in range(16):
        copies.append(pltpu.make_async_copy(
            ut3_hbm.at[u16[l] >> 3, pl.ds(u16[l] & 7, 1), :],
            urows_v.at[pl.ds(g * 16 + l, 1), :], sem))
        copies.append(pltpu.make_async_copy(
            at3_hbm.at[a16[l] >> 3, pl.ds(a16[l] & 7, 1), :],
            arows_v.at[pl.ds(g * 16 + l, 1), :], sem))
    return copies


def _sc_kernel(uid_hbm, aid_hbm, ut3_hbm, at3_hbm, out_hbm,
               uidx_v, aidx_v, urows_v, arows_v, out_v, sem):
    wid = lax.axis_index("s") * 2 + lax.axis_index("c")
    base = wid * BPW

    pltpu.sync_copy(uid_hbm.at[pl.ds(base, BPW)], uidx_v)
    pltpu.sync_copy(aid_hbm.at[pl.ds(base, BPW)], aidx_v)

    lane = lax.iota(jnp.int32, 16)

    for half in range(2):
        hb = half * HALF

        def fire(g, _):
            for c in _row_copies(ut3_hbm, at3_hbm, urows_v, arows_v,
                                 uidx_v, aidx_v, hb, g, sem):
                c.start()
            return _

        lax.fori_loop(0, NG, fire, None)

        def drain(g, _):
            for _l in range(32):
                pltpu.make_async_copy(
                    ut3_hbm.at[0, pl.ds(0, 1), :],
                    urows_v.at[pl.ds(0, 1), :], sem).wait()
            return _

        lax.fori_loop(0, NG, drain, None)

        def group_body(g, _):
            rv = g * 16 + lane
            acc = jnp.zeros((16,), jnp.float32)
            for d in range(D):
                dv = (jnp.full((16,), d, jnp.int32) + lane) & (D - 1)
                uu = plsc.load_gather(urows_v, [rv, dv])
                aa = plsc.load_gather(arows_v, [rv, dv])
                acc = acc + uu * aa
            out_v[pl.ds(hb + g * 16, 16)] = 1.0 / (1.0 + jnp.exp(-acc))
            return _

        lax.fori_loop(0, NG, group_body, None)

    pltpu.sync_copy(out_v, out_hbm.at[pl.ds(base, BPW)])


@jax.jit
def kernel(user_ids, anime_ids, user_table, anime_table):
    mesh = plsc.VectorSubcoreMesh(core_axis_name="c", subcore_axis_name="s")
    run = pl.kernel(
        _sc_kernel,
        out_type=jax.ShapeDtypeStruct((B,), jnp.float32),
        mesh=mesh,
        compiler_params=pltpu.CompilerParams(needs_layout_passes=False),
        scratch_types=[
            pltpu.VMEM((BPW,), jnp.int32),
            pltpu.VMEM((BPW,), jnp.int32),
            pltpu.VMEM((HALF, D), jnp.float32),
            pltpu.VMEM((HALF, D), jnp.float32),
            pltpu.VMEM((BPW,), jnp.float32),
            pltpu.SemaphoreType.DMA,
        ],
    )
    return run(user_ids.astype(jnp.int32), anime_ids.astype(jnp.int32),
               user_table.reshape(12500, 8, D), anime_table.reshape(12500, 8, D))


# speedup vs baseline: 1.0265x; 1.0265x over previous
import jax
import jax.numpy as jnp
from jax import lax
from jax.experimental import pallas as pl
from jax.experimental.pallas import tpu as pltpu
from jax.experimental.pallas import tpu_sc as plsc

D = 64
B = 16384

NW = 32
BPW = B // NW
HALF = BPW // 2
NG = HALF // 16


def _row_copies(ut3_hbm, at3_hbm, urows_v, arows_v, uidx_v, aidx_v, hb, g, sem):
    u16 = uidx_v[pl.ds(hb + g * 16, 16)]
    a16 = aidx_v[pl.ds(hb + g * 16, 16)]
    copies = []
    for l in range(16):
        copies.append(pltpu.make_async_copy(
            ut3_hbm.at[u16[l] >> 3, pl.ds(u16[l] & 7, 1), :],
            urows_v.at[pl.ds(g * 16 + l, 1), :], sem))
        copies.append(pltpu.make_async_copy(
            at3_hbm.at[pl.ds(a16[l], 1), :],
            arows_v.at[pl.ds(g * 16 + l, 1), :], sem))
    return copies


def _sc_kernel(uid_hbm, aid_hbm, ut3_hbm, at3_hbm, out_hbm,
               uidx_v, aidx_v, urows_v, arows_v, out_v, sem):
    wid = lax.axis_index("s") * 2 + lax.axis_index("c")
    base = wid * BPW

    pltpu.sync_copy(uid_hbm.at[pl.ds(base, BPW)], uidx_v)
    pltpu.sync_copy(aid_hbm.at[pl.ds(base, BPW)], aidx_v)

    lane = lax.iota(jnp.int32, 16)

    for half in range(2):
        hb = half * HALF

        def fire(g, _):
            for c in _row_copies(ut3_hbm, at3_hbm, urows_v, arows_v,
                                 uidx_v, aidx_v, hb, g, sem):
                c.start()
            return _

        lax.fori_loop(0, NG, fire, None)

        def drain(g, _):
            for _l in range(32):
                pltpu.make_async_copy(
                    ut3_hbm.at[0, pl.ds(0, 1), :],
                    urows_v.at[pl.ds(0, 1), :], sem).wait()
            return _

        lax.fori_loop(0, NG, drain, None)

        def group_body(g, _):
            rv = g * 16 + lane
            acc = jnp.zeros((16,), jnp.float32)
            for d in range(D):
                dv = (jnp.full((16,), d, jnp.int32) + lane) & (D - 1)
                uu = plsc.load_gather(urows_v, [rv, dv])
                aa = plsc.load_gather(arows_v, [rv, dv])
                acc = acc + uu * aa
            out_v[pl.ds(hb + g * 16, 16)] = 1.0 / (1.0 + jnp.exp(-acc))
            return _

        lax.fori_loop(0, NG, group_body, None)

    pltpu.sync_copy(out_v, out_hbm.at[pl.ds(base, BPW)])


@jax.jit
def kernel(user_ids, anime_ids, user_table, anime_table):
    mesh = plsc.VectorSubcoreMesh(core_axis_name="c", subcore_axis_name="s")
    run = pl.kernel(
        _sc_kernel,
        out_type=jax.ShapeDtypeStruct((B,), jnp.float32),
        mesh=mesh,
        compiler_params=pltpu.CompilerParams(needs_layout_passes=False),
        scratch_types=[
            pltpu.VMEM((BPW,), jnp.int32),
            pltpu.VMEM((BPW,), jnp.int32),
            pltpu.VMEM((HALF, D), jnp.float32),
            pltpu.VMEM((HALF, D), jnp.float32),
            pltpu.VMEM((BPW,), jnp.float32),
            pltpu.SemaphoreType.DMA,
        ],
    )
    return run(user_ids.astype(jnp.int32), anime_ids.astype(jnp.int32),
               user_table.reshape(12500, 8, D), anime_table)
